# baseline (speedup 1.0000x reference)
import functools

import jax
import jax.numpy as jnp
from jax import lax
from jax.experimental import pallas as pl
from jax.experimental.pallas import tpu as pltpu
from jax.experimental.pallas import tpu_sc as plsc

K = 8
V = 2048
D = 128
B = 16
T = 2048
NW = 32
P = B * T
PPW = P // NW
C = 128
NCH = PPW // C
WPB = T // PPW


def _make_kernel():
  mesh = plsc.VectorSubcoreMesh(core_axis_name="c", subcore_axis_name="s")

  @functools.partial(
      pl.kernel,
      mesh=mesh,
      out_type=jax.ShapeDtypeStruct((P, D), jnp.float32),
      scratch_types=[
          pltpu.VMEM((K, PPW), jnp.int32),
          pltpu.VMEM((C, D), jnp.float32),
          pltpu.VMEM((C, D), jnp.float32),
          pltpu.SemaphoreType.DMA,
          pltpu.SemaphoreType.DMA,
          pltpu.SemaphoreType.DMA,
          pltpu.SemaphoreType.DMA,
          pltpu.SemaphoreType.DMA,
          pltpu.SemaphoreType.DMA,
      ],
  )
  def k(codes_hbm, tab_hbm, out_hbm, idx_v, acc0, acc1, sg0, sg1, sa0, sa1,
        so0, so1):
    wid = lax.axis_index("s") * 2 + lax.axis_index("c")
    b = wid // WPB
    half = wid % WPB
    acc = (acc0, acc1)
    sg = (sg0, sg1)
    sa = (sa0, sa1)
    so = (so0, so1)
    pltpu.sync_copy(codes_hbm.at[b, :, pl.ds(half * PPW, PPW)], idx_v)

    def islice(kk, ci):
      return idx_v.at[kk, pl.ds(ci * C, C)]

    def plain(ci, p):
      pltpu.async_copy(tab_hbm.at[0].at[islice(0, ci)], acc[p], sg[p])

    def drain_plain(ci, p):
      pltpu.make_async_copy(
          tab_hbm.at[0].at[islice(0, ci)], acc[p], sg[p]).wait()

    plain(0, 0)
    for ci in range(NCH):
      p = ci % 2
      q = 1 - p
      drain_plain(ci, p)
      adds = [
          pltpu.async_copy(tab_hbm.at[kk].at[islice(kk, ci)], acc[p], sa[p],
                           add=True)
          for kk in range(1, K)
      ]
      if ci + 1 < NCH:
        if ci >= 1:
          pltpu.make_async_copy(acc[q], out_hbm.at[pl.ds(0, C)], so[q]).wait()
        plain(ci + 1, q)
      for cp in adds:
        cp.wait()
      pltpu.async_copy(acc[p], out_hbm.at[pl.ds(wid * PPW + ci * C, C)], so[p])
    pltpu.make_async_copy(
        acc[(NCH - 1) % 2],
        out_hbm.at[pl.ds(0, C)],
        so[(NCH - 1) % 2],
    ).wait()

  return k


_sc_kernel = _make_kernel()


def kernel(codes, tables):
  out = _sc_kernel(codes.astype(jnp.int32), tables)
  return out.reshape(B, T, D)


# speedup vs baseline: 1.0114x; 1.0114x over previous
import functools

import jax
import jax.numpy as jnp
from jax import lax
from jax.experimental import pallas as pl
from jax.experimental.pallas import tpu as pltpu
from jax.experimental.pallas import tpu_sc as plsc

K = 8
V = 2048
D = 128
B = 16
T = 2048
NW = 32
P = B * T
PPW = P // NW
C = 128
NCH = PPW // C
WPB = T // PPW
NB = 4


def _make_kernel():
  mesh = plsc.VectorSubcoreMesh(core_axis_name="c", subcore_axis_name="s")

  @functools.partial(
      pl.kernel,
      mesh=mesh,
      out_type=jax.ShapeDtypeStruct((P, D), jnp.float32),
      scratch_types=[pltpu.VMEM((K, PPW), jnp.int32)]
      + [pltpu.VMEM((C, D), jnp.float32)] * NB
      + [pltpu.SemaphoreType.DMA] * (3 * NB),
  )
  def k(codes_hbm, tab_hbm, out_hbm, idx_v, *bufs):
    acc = bufs[:NB]
    sg = bufs[NB:2 * NB]
    sa = bufs[2 * NB:3 * NB]
    so = bufs[3 * NB:4 * NB]
    wid = lax.axis_index("s") * 2 + lax.axis_index("c")
    b = wid // WPB
    half = wid % WPB
    pltpu.sync_copy(codes_hbm.at[b, :, pl.ds(half * PPW, PPW)], idx_v)

    def islice(kk, ci):
      return idx_v.at[kk, pl.ds(ci * C, C)]

    def plain(ci):
      p = ci % NB
      pltpu.async_copy(tab_hbm.at[0].at[islice(0, ci)], acc[p], sg[p])

    def drain_plain(ci):
      p = ci % NB
      pltpu.make_async_copy(
          tab_hbm.at[0].at[islice(0, ci)], acc[p], sg[p]).wait()

    def drain_out(ci):
      p = ci % NB
      pltpu.make_async_copy(acc[p], out_hbm.at[pl.ds(0, C)], so[p]).wait()

    for ci in range(NB - 1):
      plain(ci)
    for ci in range(NCH):
      p = ci % NB
      drain_plain(ci)
      adds = [
          pltpu.async_copy(tab_hbm.at[kk].at[islice(kk, ci)], acc[p], sa[p],
                           add=True)
          for kk in range(1, K)
      ]
      if ci + NB - 1 < NCH:
        if ci >= 1:
          drain_out(ci - 1)
        plain(ci + NB - 1)
      for cp in adds:
        cp.wait()
      pltpu.async_copy(acc[p], out_hbm.at[pl.ds(wid * PPW + ci * C, C)], so[p])
    for ci in range(max(0, NCH - NB), NCH):
      drain_out(ci)

  return k


_sc_kernel = _make_kernel()


def kernel(codes, tables):
  out = _sc_kernel(codes.astype(jnp.int32), tables)
  return out.reshape(B, T, D)
